# Initial kernel scaffold; baseline (speedup 1.0000x reference)
import jax
import jax.numpy as jnp
from jax.experimental import pallas as pl


def kernel(x, edge_index, W1, b1, W2, b2):
    raise NotImplementedError("write your pallas kernel here")


# speedup vs baseline: 16.9180x; 16.9180x over previous
import functools

import jax
import jax.numpy as jnp
from jax import lax
from jax.experimental import pallas as pl
from jax.experimental.pallas import tpu as pltpu
from jax.experimental.pallas import tpu_sc as plsc

N = 10000
NP = 10240
E = 160000
EP = 163840
D_IN = 256
D_HID = 512
D_OUT = 64
HALF = 128
CH = 128
ROWS_T = NP // 16

_mesh = plsc.VectorSubcoreMesh(core_axis_name="c", subcore_axis_name="s")


@functools.partial(
    pl.kernel, mesh=_mesh,
    out_type=jax.ShapeDtypeStruct((2, NP, HALF), jnp.float32),
    scratch_types=[
        pltpu.VMEM((EP // 32 // CH, CH), jnp.int32),
        pltpu.VMEM((CH, HALF), jnp.float32),
        pltpu.VMEM_SHARED((NP, HALF), jnp.float32),
        pltpu.SemaphoreType.DMA,
    ],
)
def _deg_kernel(dst_hbm, out_hbm, idx_v, buf_v, acc, sem):
    c = lax.axis_index("c")
    s = lax.axis_index("s")
    w = c * 16 + s
    pltpu.sync_copy(dst_hbm.at[w], idx_v)

    def fill(val):
        def frow(r, carry):
            for k in range(HALF // 16):
                buf_v[r, pl.ds(k * 16, 16)] = jnp.full((16,), val, jnp.float32)
            return carry
        lax.fori_loop(0, CH, frow, 0)

    fill(0.0)
    for i in range(ROWS_T // CH):
        pltpu.sync_copy(buf_v, acc.at[pl.ds(s * ROWS_T + i * CH, CH)])
    fill(1.0)
    plsc.subcore_barrier()

    def body(j, carry):
        pltpu.sync_copy(buf_v, acc.at[idx_v.at[j]], add=True)
        return carry

    lax.fori_loop(0, EP // 32 // CH, body, 0)
    plsc.subcore_barrier()
    pltpu.sync_copy(acc.at[pl.ds(s * ROWS_T, ROWS_T)],
                    out_hbm.at[c, pl.ds(s * ROWS_T, ROWS_T)])


@functools.partial(
    pl.kernel, mesh=_mesh,
    out_type=jax.ShapeDtypeStruct((2, NP, HALF), jnp.float32),
    scratch_types=[
        pltpu.VMEM((EP // 16 // CH, CH), jnp.int32),
        pltpu.VMEM((EP // 16 // CH, CH), jnp.int32),
        pltpu.VMEM((CH, HALF), jnp.float32),
        pltpu.VMEM_SHARED((NP, HALF), jnp.float32),
        pltpu.SemaphoreType.DMA,
    ],
)
def _prop1_kernel(xs_hbm, src_hbm, dst_hbm, out_hbm, src_v, dst_v, rows_v, acc, sem):
    c = lax.axis_index("c")
    s = lax.axis_index("s")
    pltpu.sync_copy(src_hbm.at[c, s], src_v)
    pltpu.sync_copy(dst_hbm.at[s], dst_v)
    pltpu.sync_copy(xs_hbm.at[pl.ds(c * NP + s * ROWS_T, ROWS_T)],
                    acc.at[pl.ds(s * ROWS_T, ROWS_T)])
    plsc.subcore_barrier()

    def body(j, carry):
        pltpu.async_copy(xs_hbm.at[src_v.at[j]], rows_v, sem).wait()
        pltpu.sync_copy(rows_v, acc.at[dst_v.at[j]], add=True)
        return carry

    lax.fori_loop(0, EP // 16 // CH, body, 0)
    plsc.subcore_barrier()
    pltpu.sync_copy(acc.at[pl.ds(s * ROWS_T, ROWS_T)],
                    out_hbm.at[c, pl.ds(s * ROWS_T, ROWS_T)])


@functools.partial(
    pl.kernel, mesh=_mesh,
    out_type=jax.ShapeDtypeStruct((2, NP, HALF), jnp.float32),
    scratch_types=[
        pltpu.VMEM((EP // 32 // CH, CH), jnp.int32),
        pltpu.VMEM((EP // 32 // CH, CH), jnp.int32),
        pltpu.VMEM((CH, HALF), jnp.float32),
        pltpu.VMEM_SHARED((NP, HALF), jnp.float32),
        pltpu.SemaphoreType.DMA,
    ],
)
def _prop2_kernel(zs_hbm, init_hbm, src_hbm, dst_hbm, out_hbm,
                  src_v, dst_v, rows_v, acc, sem):
    c = lax.axis_index("c")
    s = lax.axis_index("s")
    w = c * 16 + s
    pltpu.sync_copy(src_hbm.at[w], src_v)
    pltpu.sync_copy(dst_hbm.at[w], dst_v)
    pltpu.sync_copy(init_hbm.at[c, pl.ds(s * ROWS_T, ROWS_T)],
                    acc.at[pl.ds(s * ROWS_T, ROWS_T)])
    plsc.subcore_barrier()

    def body(j, carry):
        pltpu.async_copy(zs_hbm.at[src_v.at[j]], rows_v, sem).wait()
        pltpu.sync_copy(rows_v, acc.at[dst_v.at[j]], add=True)
        return carry

    lax.fori_loop(0, EP // 32 // CH, body, 0)
    plsc.subcore_barrier()
    pltpu.sync_copy(acc.at[pl.ds(s * ROWS_T, ROWS_T)],
                    out_hbm.at[c, pl.ds(s * ROWS_T, ROWS_T)])


_RB = 1024


def _tc1_body(degp_ref, x_ref, dis_ref, xs_ref):
    deg = degp_ref[0, :, 0:1] + degp_ref[1, :, 0:1] + 1.0
    dis = lax.rsqrt(deg)
    dis_ref[...] = dis
    xs_ref[0] = x_ref[:, 0:HALF] * dis
    xs_ref[1] = x_ref[:, HALF:D_IN] * dis


def _tc1(degp, x_pad):
    grid = (NP // _RB,)
    return pl.pallas_call(
        _tc1_body,
        grid=grid,
        in_specs=[
            pl.BlockSpec((2, _RB, HALF), lambda i: (0, i, 0)),
            pl.BlockSpec((_RB, D_IN), lambda i: (i, 0)),
        ],
        out_specs=[
            pl.BlockSpec((_RB, 1), lambda i: (i, 0)),
            pl.BlockSpec((2, _RB, HALF), lambda i: (0, i, 0)),
        ],
        out_shape=[
            jax.ShapeDtypeStruct((NP, 1), jnp.float32),
            jax.ShapeDtypeStruct((2, NP, HALF), jnp.float32),
        ],
    )(degp, x_pad)


def _tc2_body(p1_ref, dis_ref, w1_ref, b1_ref, w2_ref, zs_ref):
    y = jnp.dot(p1_ref[0], w1_ref[0], preferred_element_type=jnp.float32)
    y = y + jnp.dot(p1_ref[1], w1_ref[1], preferred_element_type=jnp.float32)
    h = jnp.maximum(y * dis_ref[...] + b1_ref[...], 0.0)
    zs_ref[:, 0:D_OUT] = jnp.dot(h, w2_ref[...],
                                 preferred_element_type=jnp.float32) * dis_ref[...]
    zs_ref[:, D_OUT:HALF] = jnp.zeros((_RB, HALF - D_OUT), jnp.float32)


def _tc2(p1, dis, W1r, b1r, W2):
    grid = (NP // _RB,)
    return pl.pallas_call(
        _tc2_body,
        grid=grid,
        in_specs=[
            pl.BlockSpec((2, _RB, HALF), lambda i: (0, i, 0)),
            pl.BlockSpec((_RB, 1), lambda i: (i, 0)),
            pl.BlockSpec((2, HALF, D_HID), lambda i: (0, 0, 0)),
            pl.BlockSpec((1, D_HID), lambda i: (0, 0)),
            pl.BlockSpec((D_HID, D_OUT), lambda i: (0, 0)),
        ],
        out_specs=pl.BlockSpec((_RB, HALF), lambda i: (i, 0)),
        out_shape=jax.ShapeDtypeStruct((NP, HALF), jnp.float32),
    )(p1, dis, W1r, b1r, W2)


def _tc3_body(p2_ref, dis_ref, b2_ref, out_ref):
    out_ref[...] = ((p2_ref[0, :, 0:D_OUT] + p2_ref[1, :, 0:D_OUT])
                    * dis_ref[...] + b2_ref[...])


def _tc3(p2, dis, b2r):
    grid = (NP // _RB,)
    return pl.pallas_call(
        _tc3_body,
        grid=grid,
        in_specs=[
            pl.BlockSpec((2, _RB, HALF), lambda i: (0, i, 0)),
            pl.BlockSpec((_RB, 1), lambda i: (i, 0)),
            pl.BlockSpec((1, D_OUT), lambda i: (0, 0)),
        ],
        out_specs=pl.BlockSpec((_RB, D_OUT), lambda i: (i, 0)),
        out_shape=jax.ShapeDtypeStruct((NP, D_OUT), jnp.float32),
    )(p2, dis, b2r)


def kernel(x, edge_index, W1, b1, W2, b2):
    src = edge_index[0].astype(jnp.int32)
    dst = edge_index[1].astype(jnp.int32)

    pad_ids = N + (jnp.arange(EP - E, dtype=jnp.int32) % 16)
    src_p = jnp.concatenate([src, pad_ids])
    dst_p = jnp.concatenate([dst, pad_ids])

    dst16 = dst_p.reshape(16, EP // 16 // CH, CH)
    src16o = jnp.stack([src_p, src_p + NP]).reshape(2, 16, EP // 16 // CH, CH)
    src32 = src_p.reshape(32, EP // 32 // CH, CH)
    dst32 = dst_p.reshape(32, EP // 32 // CH, CH)

    x_pad = jnp.pad(x, ((0, NP - N), (0, 0)))

    degp = _deg_kernel(dst32)

    dis, xs = _tc1(degp, x_pad)
    xs_flat = xs.reshape(2 * NP, HALF)

    p1 = _prop1_kernel(xs_flat, src16o, dst16)

    W1r = W1.reshape(2, HALF, D_HID)
    zs = _tc2(p1, dis, W1r, b1.reshape(1, D_HID), W2)

    init2 = jnp.concatenate(
        [zs[None], jnp.zeros((1, NP, HALF), jnp.float32)], axis=0)
    p2 = _prop2_kernel(zs, init2, src32, dst32)

    out = _tc3(p2, dis, b2.reshape(1, D_OUT))
    return out[:N]


# speedup vs baseline: 20.6380x; 1.2199x over previous
import functools

import jax
import jax.numpy as jnp
from jax import lax
from jax.experimental import pallas as pl
from jax.experimental.pallas import tpu as pltpu
from jax.experimental.pallas import tpu_sc as plsc

N = 10000
NP = 10240
E = 160000
EP = 163840
D_IN = 256
D_HID = 512
D_OUT = 64
HALF = 128
CH = 128
ROWS_T = NP // 16

_mesh = plsc.VectorSubcoreMesh(core_axis_name="c", subcore_axis_name="s")


@functools.partial(
    pl.kernel, mesh=_mesh,
    out_type=jax.ShapeDtypeStruct((2, NP, HALF), jnp.float32),
    scratch_types=[
        pltpu.VMEM((EP // 32 // CH, CH), jnp.int32),
        pltpu.VMEM((CH, HALF), jnp.float32),
        pltpu.VMEM_SHARED((NP, HALF), jnp.float32),
        pltpu.SemaphoreType.DMA,
    ],
)
def _deg_kernel(dst_hbm, out_hbm, idx_v, buf_v, acc, sem):
    c = lax.axis_index("c")
    s = lax.axis_index("s")
    w = c * 16 + s
    pltpu.sync_copy(dst_hbm.at[w], idx_v)

    def fill(val):
        def frow(r, carry):
            for k in range(HALF // 16):
                buf_v[r, pl.ds(k * 16, 16)] = jnp.full((16,), val, jnp.float32)
            return carry
        lax.fori_loop(0, CH, frow, 0)

    fill(0.0)
    for i in range(ROWS_T // CH):
        pltpu.sync_copy(buf_v, acc.at[pl.ds(s * ROWS_T + i * CH, CH)])
    fill(1.0)
    plsc.subcore_barrier()

    def body(j, carry):
        pltpu.sync_copy(buf_v, acc.at[idx_v.at[j]], add=True)
        return carry

    lax.fori_loop(0, EP // 32 // CH, body, 0)
    plsc.subcore_barrier()
    pltpu.sync_copy(acc.at[pl.ds(s * ROWS_T, ROWS_T)],
                    out_hbm.at[c, pl.ds(s * ROWS_T, ROWS_T)])


NBUF = 2
NIDX = 4


def _edge_pipeline(table_hbm, src_row, dst_row, srcr, dstr, bufs,
                   isems, gsems, ssems, acc, n_chunks):
    for k in range(NBUF):
        pltpu.async_copy(src_row(k), srcr.at[k], isems[k])
        pltpu.async_copy(dst_row(k), dstr.at[k], isems[k])

    def chunk(j, b, q, qn):
        @pl.when(j >= NBUF)
        def _():
            pltpu.make_async_copy(bufs[b], acc.at[dstr.at[q]], ssems[b]).wait()

        @pl.when(j + NBUF < n_chunks)
        def _():
            pltpu.async_copy(src_row(j + NBUF), srcr.at[qn], isems[qn])
            pltpu.async_copy(dst_row(j + NBUF), dstr.at[qn], isems[qn])

        pltpu.make_async_copy(src_row(j), srcr.at[q], isems[q]).wait()
        pltpu.make_async_copy(dst_row(j), dstr.at[q], isems[q]).wait()
        pltpu.async_copy(table_hbm.at[srcr.at[q]], bufs[b], gsems[b])
        pltpu.make_async_copy(table_hbm.at[srcr.at[q]], bufs[b], gsems[b]).wait()
        pltpu.async_copy(bufs[b], acc.at[dstr.at[q]], ssems[b], add=True)

    def group(g, carry):
        for u in range(NIDX):
            chunk(g * NIDX + u, u % NBUF, u, (u + NBUF) % NIDX)
        return carry

    lax.fori_loop(0, n_chunks // NIDX, group, 0)
    for b in range(NBUF):
        pltpu.make_async_copy(bufs[b], acc.at[pl.ds(0, CH)], ssems[b]).wait()


_PIPE_SCRATCH = (
    [pltpu.VMEM((NIDX, CH), jnp.int32),
     pltpu.VMEM((NIDX, CH), jnp.int32)]
    + [pltpu.VMEM((CH, HALF), jnp.float32)] * NBUF
    + [pltpu.VMEM_SHARED((NP, HALF), jnp.float32)]
    + [pltpu.SemaphoreType.DMA] * (NIDX + 2 * NBUF)
)


def _unpack_pipe(rest):
    srcr, dstr = rest[0], rest[1]
    bufs = rest[2:2 + NBUF]
    acc = rest[2 + NBUF]
    sems = rest[3 + NBUF:]
    isems = sems[:NIDX]
    gsems = sems[NIDX:NIDX + NBUF]
    ssems = sems[NIDX + NBUF:]
    return srcr, dstr, bufs, acc, isems, gsems, ssems


@functools.partial(
    pl.kernel, mesh=_mesh,
    out_type=jax.ShapeDtypeStruct((2, NP, HALF), jnp.float32),
    scratch_types=list(_PIPE_SCRATCH),
)
def _prop1_kernel(xs_hbm, src_hbm, dst_hbm, out_hbm, *rest):
    srcr, dstr, bufs, acc, isems, gsems, ssems = _unpack_pipe(rest)
    c = lax.axis_index("c")
    s = lax.axis_index("s")
    pltpu.sync_copy(xs_hbm.at[pl.ds(c * NP + s * ROWS_T, ROWS_T)],
                    acc.at[pl.ds(s * ROWS_T, ROWS_T)])
    plsc.subcore_barrier()
    _edge_pipeline(xs_hbm,
                   lambda j: src_hbm.at[c, s, j],
                   lambda j: dst_hbm.at[s, j],
                   srcr, dstr, bufs, isems, gsems, ssems, acc, EP // 16 // CH)
    plsc.subcore_barrier()
    pltpu.sync_copy(acc.at[pl.ds(s * ROWS_T, ROWS_T)],
                    out_hbm.at[c, pl.ds(s * ROWS_T, ROWS_T)])


@functools.partial(
    pl.kernel, mesh=_mesh,
    out_type=jax.ShapeDtypeStruct((2, NP, HALF), jnp.float32),
    scratch_types=list(_PIPE_SCRATCH),
)
def _prop2_kernel(zs_hbm, init_hbm, src_hbm, dst_hbm, out_hbm, *rest):
    srcr, dstr, bufs, acc, isems, gsems, ssems = _unpack_pipe(rest)
    c = lax.axis_index("c")
    s = lax.axis_index("s")
    w = c * 16 + s
    pltpu.sync_copy(init_hbm.at[c, pl.ds(s * ROWS_T, ROWS_T)],
                    acc.at[pl.ds(s * ROWS_T, ROWS_T)])
    plsc.subcore_barrier()
    _edge_pipeline(zs_hbm,
                   lambda j: src_hbm.at[w, j],
                   lambda j: dst_hbm.at[w, j],
                   srcr, dstr, bufs, isems, gsems, ssems, acc, EP // 32 // CH)
    plsc.subcore_barrier()
    pltpu.sync_copy(acc.at[pl.ds(s * ROWS_T, ROWS_T)],
                    out_hbm.at[c, pl.ds(s * ROWS_T, ROWS_T)])


_RB = 1024


def _tc1_body(degp_ref, x_ref, dis_ref, xs_ref):
    deg = degp_ref[0, :, 0:1] + degp_ref[1, :, 0:1] + 1.0
    dis = lax.rsqrt(deg)
    dis_ref[...] = dis
    xs_ref[0] = x_ref[:, 0:HALF] * dis
    xs_ref[1] = x_ref[:, HALF:D_IN] * dis


def _tc1(degp, x_pad):
    grid = (NP // _RB,)
    return pl.pallas_call(
        _tc1_body,
        grid=grid,
        in_specs=[
            pl.BlockSpec((2, _RB, HALF), lambda i: (0, i, 0)),
            pl.BlockSpec((_RB, D_IN), lambda i: (i, 0)),
        ],
        out_specs=[
            pl.BlockSpec((_RB, 1), lambda i: (i, 0)),
            pl.BlockSpec((2, _RB, HALF), lambda i: (0, i, 0)),
        ],
        out_shape=[
            jax.ShapeDtypeStruct((NP, 1), jnp.float32),
            jax.ShapeDtypeStruct((2, NP, HALF), jnp.float32),
        ],
    )(degp, x_pad)


def _tc2_body(p1_ref, dis_ref, w1_ref, b1_ref, w2_ref, zs_ref):
    y = jnp.dot(p1_ref[0], w1_ref[0], preferred_element_type=jnp.float32)
    y = y + jnp.dot(p1_ref[1], w1_ref[1], preferred_element_type=jnp.float32)
    h = jnp.maximum(y * dis_ref[...] + b1_ref[...], 0.0)
    zs_ref[:, 0:D_OUT] = jnp.dot(h, w2_ref[...],
                                 preferred_element_type=jnp.float32) * dis_ref[...]
    zs_ref[:, D_OUT:HALF] = jnp.zeros((_RB, HALF - D_OUT), jnp.float32)


def _tc2(p1, dis, W1r, b1r, W2):
    grid = (NP // _RB,)
    return pl.pallas_call(
        _tc2_body,
        grid=grid,
        in_specs=[
            pl.BlockSpec((2, _RB, HALF), lambda i: (0, i, 0)),
            pl.BlockSpec((_RB, 1), lambda i: (i, 0)),
            pl.BlockSpec((2, HALF, D_HID), lambda i: (0, 0, 0)),
            pl.BlockSpec((1, D_HID), lambda i: (0, 0)),
            pl.BlockSpec((D_HID, D_OUT), lambda i: (0, 0)),
        ],
        out_specs=pl.BlockSpec((_RB, HALF), lambda i: (i, 0)),
        out_shape=jax.ShapeDtypeStruct((NP, HALF), jnp.float32),
    )(p1, dis, W1r, b1r, W2)


def _tc3_body(p2_ref, dis_ref, b2_ref, out_ref):
    out_ref[...] = ((p2_ref[0, :, 0:D_OUT] + p2_ref[1, :, 0:D_OUT])
                    * dis_ref[...] + b2_ref[...])


def _tc3(p2, dis, b2r):
    grid = (NP // _RB,)
    return pl.pallas_call(
        _tc3_body,
        grid=grid,
        in_specs=[
            pl.BlockSpec((2, _RB, HALF), lambda i: (0, i, 0)),
            pl.BlockSpec((_RB, 1), lambda i: (i, 0)),
            pl.BlockSpec((1, D_OUT), lambda i: (0, 0)),
        ],
        out_specs=pl.BlockSpec((_RB, D_OUT), lambda i: (i, 0)),
        out_shape=jax.ShapeDtypeStruct((NP, D_OUT), jnp.float32),
    )(p2, dis, b2r)


def kernel(x, edge_index, W1, b1, W2, b2):
    src = edge_index[0].astype(jnp.int32)
    dst = edge_index[1].astype(jnp.int32)

    pad_ids = N + (jnp.arange(EP - E, dtype=jnp.int32) % 16)
    src_p = jnp.concatenate([src, pad_ids])
    dst_p = jnp.concatenate([dst, pad_ids])

    dst16 = dst_p.reshape(16, EP // 16 // CH, CH)
    src16o = jnp.stack([src_p, src_p + NP]).reshape(2, 16, EP // 16 // CH, CH)
    src32 = src_p.reshape(32, EP // 32 // CH, CH)
    dst32 = dst_p.reshape(32, EP // 32 // CH, CH)

    x_pad = jnp.pad(x, ((0, NP - N), (0, 0)))

    degp = _deg_kernel(dst32)

    dis, xs = _tc1(degp, x_pad)
    xs_flat = xs.reshape(2 * NP, HALF)

    p1 = _prop1_kernel(xs_flat, src16o, dst16)

    W1r = W1.reshape(2, HALF, D_HID)
    zs = _tc2(p1, dis, W1r, b1.reshape(1, D_HID), W2)

    init2 = jnp.concatenate(
        [zs[None], jnp.zeros((1, NP, HALF), jnp.float32)], axis=0)
    p2 = _prop2_kernel(zs, init2, src32, dst32)

    out = _tc3(p2, dis, b2.reshape(1, D_OUT))
    return out[:N]


# speedup vs baseline: 20.8065x; 1.0082x over previous
import functools

import jax
import jax.numpy as jnp
from jax import lax
from jax.experimental import pallas as pl
from jax.experimental.pallas import tpu as pltpu
from jax.experimental.pallas import tpu_sc as plsc

N = 10000
NP = 10240
E = 160000
EP = 163840
D_IN = 256
D_HID = 512
D_OUT = 64
HALF = 128
CH = 128
ROWS_T = NP // 16

_mesh = plsc.VectorSubcoreMesh(core_axis_name="c", subcore_axis_name="s")


@functools.partial(
    pl.kernel, mesh=_mesh,
    out_type=jax.ShapeDtypeStruct((2, NP, HALF), jnp.float32),
    scratch_types=[
        pltpu.VMEM((EP // 32 // CH, CH), jnp.int32),
        pltpu.VMEM((CH, HALF), jnp.float32),
        pltpu.VMEM_SHARED((NP, HALF), jnp.float32),
        pltpu.SemaphoreType.DMA,
    ],
)
def _deg_kernel(dst_hbm, out_hbm, idx_v, buf_v, acc, sem):
    c = lax.axis_index("c")
    s = lax.axis_index("s")
    w = c * 16 + s
    pltpu.sync_copy(dst_hbm.at[w], idx_v)

    def fill(val):
        def frow(r, carry):
            for k in range(HALF // 16):
                buf_v[r, pl.ds(k * 16, 16)] = jnp.full((16,), val, jnp.float32)
            return carry
        lax.fori_loop(0, CH, frow, 0)

    fill(0.0)
    for i in range(ROWS_T // CH):
        pltpu.sync_copy(buf_v, acc.at[pl.ds(s * ROWS_T + i * CH, CH)])
    fill(1.0)
    plsc.subcore_barrier()

    def body(j, carry):
        pltpu.async_copy(buf_v, acc.at[idx_v.at[j]], sem, add=True)
        return carry

    lax.fori_loop(0, EP // 32 // CH, body, 0)

    def drain(j, carry):
        pltpu.make_async_copy(buf_v, acc.at[idx_v.at[j]], sem).wait()
        return carry

    lax.fori_loop(0, EP // 32 // CH, drain, 0)
    plsc.subcore_barrier()
    pltpu.sync_copy(acc.at[pl.ds(s * ROWS_T, ROWS_T)],
                    out_hbm.at[c, pl.ds(s * ROWS_T, ROWS_T)])


NBUF = 2
NIDX = 4


def _edge_pipeline(table_hbm, src_row, dst_row, srcr, dstr, bufs,
                   isems, gsems, ssems, acc, n_chunks):
    for k in range(NBUF):
        pltpu.async_copy(src_row(k), srcr.at[k], isems[k])
        pltpu.async_copy(dst_row(k), dstr.at[k], isems[k])

    def chunk(j, b, q, qn):
        @pl.when(j >= NBUF)
        def _():
            pltpu.make_async_copy(bufs[b], acc.at[dstr.at[q]], ssems[b]).wait()

        @pl.when(j + NBUF < n_chunks)
        def _():
            pltpu.async_copy(src_row(j + NBUF), srcr.at[qn], isems[qn])
            pltpu.async_copy(dst_row(j + NBUF), dstr.at[qn], isems[qn])

        pltpu.make_async_copy(src_row(j), srcr.at[q], isems[q]).wait()
        pltpu.make_async_copy(dst_row(j), dstr.at[q], isems[q]).wait()
        pltpu.async_copy(table_hbm.at[srcr.at[q]], bufs[b], gsems[b])
        pltpu.make_async_copy(table_hbm.at[srcr.at[q]], bufs[b], gsems[b]).wait()
        pltpu.async_copy(bufs[b], acc.at[dstr.at[q]], ssems[b], add=True)

    def group(g, carry):
        for u in range(NIDX):
            chunk(g * NIDX + u, u % NBUF, u, (u + NBUF) % NIDX)
        return carry

    lax.fori_loop(0, n_chunks // NIDX, group, 0)
    for b in range(NBUF):
        pltpu.make_async_copy(bufs[b], acc.at[pl.ds(0, CH)], ssems[b]).wait()


_PIPE_SCRATCH = (
    [pltpu.VMEM((NIDX, CH), jnp.int32),
     pltpu.VMEM((NIDX, CH), jnp.int32)]
    + [pltpu.VMEM((CH, HALF), jnp.float32)] * NBUF
    + [pltpu.VMEM_SHARED((NP, HALF), jnp.float32)]
    + [pltpu.SemaphoreType.DMA] * (NIDX + 2 * NBUF)
)


def _unpack_pipe(rest):
    srcr, dstr = rest[0], rest[1]
    bufs = rest[2:2 + NBUF]
    acc = rest[2 + NBUF]
    sems = rest[3 + NBUF:]
    isems = sems[:NIDX]
    gsems = sems[NIDX:NIDX + NBUF]
    ssems = sems[NIDX + NBUF:]
    return srcr, dstr, bufs, acc, isems, gsems, ssems


@functools.partial(
    pl.kernel, mesh=_mesh,
    out_type=jax.ShapeDtypeStruct((2, NP, HALF), jnp.float32),
    scratch_types=list(_PIPE_SCRATCH),
)
def _prop1_kernel(xs_hbm, src_hbm, dst_hbm, out_hbm, *rest):
    srcr, dstr, bufs, acc, isems, gsems, ssems = _unpack_pipe(rest)
    c = lax.axis_index("c")
    s = lax.axis_index("s")
    pltpu.sync_copy(xs_hbm.at[pl.ds(c * NP + s * ROWS_T, ROWS_T)],
                    acc.at[pl.ds(s * ROWS_T, ROWS_T)])
    plsc.subcore_barrier()
    _edge_pipeline(xs_hbm,
                   lambda j: src_hbm.at[c, s, j],
                   lambda j: dst_hbm.at[s, j],
                   srcr, dstr, bufs, isems, gsems, ssems, acc, EP // 16 // CH)
    plsc.subcore_barrier()
    pltpu.sync_copy(acc.at[pl.ds(s * ROWS_T, ROWS_T)],
                    out_hbm.at[c, pl.ds(s * ROWS_T, ROWS_T)])


@functools.partial(
    pl.kernel, mesh=_mesh,
    out_type=jax.ShapeDtypeStruct((2, NP, HALF), jnp.float32),
    scratch_types=list(_PIPE_SCRATCH),
)
def _prop2_kernel(zs_hbm, init_hbm, src_hbm, dst_hbm, out_hbm, *rest):
    srcr, dstr, bufs, acc, isems, gsems, ssems = _unpack_pipe(rest)
    c = lax.axis_index("c")
    s = lax.axis_index("s")
    w = c * 16 + s
    pltpu.sync_copy(init_hbm.at[c, pl.ds(s * ROWS_T, ROWS_T)],
                    acc.at[pl.ds(s * ROWS_T, ROWS_T)])
    plsc.subcore_barrier()
    _edge_pipeline(zs_hbm,
                   lambda j: src_hbm.at[w, j],
                   lambda j: dst_hbm.at[w, j],
                   srcr, dstr, bufs, isems, gsems, ssems, acc, EP // 32 // CH)
    plsc.subcore_barrier()
    pltpu.sync_copy(acc.at[pl.ds(s * ROWS_T, ROWS_T)],
                    out_hbm.at[c, pl.ds(s * ROWS_T, ROWS_T)])


_RB = 1024


def _tc1_body(degp_ref, x_ref, dis_ref, xs_ref):
    deg = degp_ref[0, :, 0:1] + degp_ref[1, :, 0:1] + 1.0
    dis = lax.rsqrt(deg)
    dis_ref[...] = dis
    xs_ref[0] = x_ref[:, 0:HALF] * dis
    xs_ref[1] = x_ref[:, HALF:D_IN] * dis


def _tc1(degp, x_pad):
    grid = (NP // _RB,)
    return pl.pallas_call(
        _tc1_body,
        grid=grid,
        in_specs=[
            pl.BlockSpec((2, _RB, HALF), lambda i: (0, i, 0)),
            pl.BlockSpec((_RB, D_IN), lambda i: (i, 0)),
        ],
        out_specs=[
            pl.BlockSpec((_RB, 1), lambda i: (i, 0)),
            pl.BlockSpec((2, _RB, HALF), lambda i: (0, i, 0)),
        ],
        out_shape=[
            jax.ShapeDtypeStruct((NP, 1), jnp.float32),
            jax.ShapeDtypeStruct((2, NP, HALF), jnp.float32),
        ],
    )(degp, x_pad)


def _tc2_body(p1_ref, dis_ref, w1_ref, b1_ref, w2_ref, zs_ref, init_ref):
    y = jnp.dot(p1_ref[0], w1_ref[0], preferred_element_type=jnp.float32)
    y = y + jnp.dot(p1_ref[1], w1_ref[1], preferred_element_type=jnp.float32)
    h = jnp.maximum(y * dis_ref[...] + b1_ref[...], 0.0)
    zs = jnp.dot(h, w2_ref[...],
                 preferred_element_type=jnp.float32) * dis_ref[...]
    zfull = jnp.concatenate(
        [zs, jnp.zeros((_RB, HALF - D_OUT), jnp.float32)], axis=1)
    zs_ref[...] = zfull
    init_ref[0] = zfull
    init_ref[1] = jnp.zeros((_RB, HALF), jnp.float32)


def _tc2(p1, dis, W1r, b1r, W2):
    grid = (NP // _RB,)
    return pl.pallas_call(
        _tc2_body,
        grid=grid,
        in_specs=[
            pl.BlockSpec((2, _RB, HALF), lambda i: (0, i, 0)),
            pl.BlockSpec((_RB, 1), lambda i: (i, 0)),
            pl.BlockSpec((2, HALF, D_HID), lambda i: (0, 0, 0)),
            pl.BlockSpec((1, D_HID), lambda i: (0, 0)),
            pl.BlockSpec((D_HID, D_OUT), lambda i: (0, 0)),
        ],
        out_specs=[
            pl.BlockSpec((_RB, HALF), lambda i: (i, 0)),
            pl.BlockSpec((2, _RB, HALF), lambda i: (0, i, 0)),
        ],
        out_shape=[
            jax.ShapeDtypeStruct((NP, HALF), jnp.float32),
            jax.ShapeDtypeStruct((2, NP, HALF), jnp.float32),
        ],
    )(p1, dis, W1r, b1r, W2)


def _tc3_body(p2_ref, dis_ref, b2_ref, out_ref):
    out_ref[...] = ((p2_ref[0, :, 0:D_OUT] + p2_ref[1, :, 0:D_OUT])
                    * dis_ref[...] + b2_ref[...])


def _tc3(p2, dis, b2r):
    grid = (NP // _RB,)
    return pl.pallas_call(
        _tc3_body,
        grid=grid,
        in_specs=[
            pl.BlockSpec((2, _RB, HALF), lambda i: (0, i, 0)),
            pl.BlockSpec((_RB, 1), lambda i: (i, 0)),
            pl.BlockSpec((1, D_OUT), lambda i: (0, 0)),
        ],
        out_specs=pl.BlockSpec((_RB, D_OUT), lambda i: (i, 0)),
        out_shape=jax.ShapeDtypeStruct((NP, D_OUT), jnp.float32),
    )(p2, dis, b2r)


def kernel(x, edge_index, W1, b1, W2, b2):
    src = edge_index[0].astype(jnp.int32)
    dst = edge_index[1].astype(jnp.int32)

    pad_ids = N + (jnp.arange(EP - E, dtype=jnp.int32) % 16)
    src_p = jnp.concatenate([src, pad_ids])
    dst_p = jnp.concatenate([dst, pad_ids])

    dst16 = dst_p.reshape(16, EP // 16 // CH, CH)
    src16o = jnp.stack([src_p, src_p + NP]).reshape(2, 16, EP // 16 // CH, CH)
    src32 = src_p.reshape(32, EP // 32 // CH, CH)
    dst32 = dst_p.reshape(32, EP // 32 // CH, CH)

    x_pad = jnp.pad(x, ((0, NP - N), (0, 0)))

    degp = _deg_kernel(dst32)

    dis, xs = _tc1(degp, x_pad)
    xs_flat = xs.reshape(2 * NP, HALF)

    p1 = _prop1_kernel(xs_flat, src16o, dst16)

    W1r = W1.reshape(2, HALF, D_HID)
    zs, init2 = _tc2(p1, dis, W1r, b1.reshape(1, D_HID), W2)

    p2 = _prop2_kernel(zs, init2, src32, dst32)

    out = _tc3(p2, dis, b2.reshape(1, D_OUT))
    return out[:N]


# speedup vs baseline: 21.0127x; 1.0099x over previous
import functools

import jax
import jax.numpy as jnp
from jax import lax
from jax.experimental import pallas as pl
from jax.experimental.pallas import tpu as pltpu
from jax.experimental.pallas import tpu_sc as plsc

N = 10000
NP = 10240
E = 160000
EP = 163840
D_IN = 256
D_HID = 512
D_OUT = 64
HALF = 128
CH = 128
ROWS_T = NP // 16

_mesh = plsc.VectorSubcoreMesh(core_axis_name="c", subcore_axis_name="s")


def _fill(buf_v, val):
    def frow(r, carry):
        for k in range(HALF // 16):
            buf_v[r, pl.ds(k * 16, 16)] = jnp.full((16,), val, jnp.float32)
        return carry
    lax.fori_loop(0, CH, frow, 0)


@functools.partial(
    pl.kernel, mesh=_mesh,
    out_type=jax.ShapeDtypeStruct((2, NP, HALF), jnp.float32),
    scratch_types=[
        pltpu.VMEM((EP // 32 // CH, CH), jnp.int32),
        pltpu.VMEM((CH, HALF), jnp.float32),
        pltpu.VMEM_SHARED((NP, HALF), jnp.float32),
        pltpu.SemaphoreType.DMA,
    ],
)
def _deg_kernel(dst_hbm, out_hbm, idx_v, buf_v, acc, sem):
    c = lax.axis_index("c")
    s = lax.axis_index("s")
    w = c * 16 + s
    pltpu.sync_copy(dst_hbm.at[w], idx_v)

    _fill(buf_v, 0.0)
    for i in range(ROWS_T // CH):
        pltpu.sync_copy(buf_v, acc.at[pl.ds(s * ROWS_T + i * CH, CH)])
    _fill(buf_v, 1.0)
    plsc.subcore_barrier()

    def body(j, carry):
        pltpu.async_copy(buf_v, acc.at[idx_v.at[j]], sem, add=True)
        return carry

    lax.fori_loop(0, EP // 32 // CH, body, 0)

    def drain(j, carry):
        pltpu.make_async_copy(buf_v, acc.at[idx_v.at[j]], sem).wait()
        return carry

    lax.fori_loop(0, EP // 32 // CH, drain, 0)
    plsc.subcore_barrier()
    pltpu.sync_copy(acc.at[pl.ds(s * ROWS_T, ROWS_T)],
                    out_hbm.at[c, pl.ds(s * ROWS_T, ROWS_T)])


NBUF = 2
NIDX = 4


def _edge_pipeline(table_hbm, src_row, dst_row, srcr, dstr, bufs,
                   isems, gsems, ssems, acc, n_chunks, src_off=None):
    for k in range(NBUF):
        pltpu.async_copy(src_row(k), srcr.at[k], isems[k])
        pltpu.async_copy(dst_row(k), dstr.at[k], isems[k])

    def chunk(j, b, q, qn):
        @pl.when(j >= NBUF)
        def _():
            pltpu.make_async_copy(bufs[b], acc.at[dstr.at[q]], ssems[b]).wait()

        @pl.when(j + NBUF < n_chunks)
        def _():
            pltpu.async_copy(src_row(j + NBUF), srcr.at[qn], isems[qn])
            pltpu.async_copy(dst_row(j + NBUF), dstr.at[qn], isems[qn])

        pltpu.make_async_copy(src_row(j), srcr.at[q], isems[q]).wait()
        pltpu.make_async_copy(dst_row(j), dstr.at[q], isems[q]).wait()
        if src_off is not None:
            for k in range(CH // 16):
                srcr[q, pl.ds(16 * k, 16)] = srcr[q, pl.ds(16 * k, 16)] + src_off
        pltpu.async_copy(table_hbm.at[srcr.at[q]], bufs[b], gsems[b])
        pltpu.make_async_copy(table_hbm.at[srcr.at[q]], bufs[b], gsems[b]).wait()
        pltpu.async_copy(bufs[b], acc.at[dstr.at[q]], ssems[b], add=True)

    def group(g, carry):
        for u in range(NIDX):
            chunk(g * NIDX + u, u % NBUF, u, (u + NBUF) % NIDX)
        return carry

    lax.fori_loop(0, n_chunks // NIDX, group, 0)
    for b in range(NBUF):
        pltpu.make_async_copy(bufs[b], acc.at[pl.ds(0, CH)], ssems[b]).wait()


_PIPE_SCRATCH = (
    [pltpu.VMEM((NIDX, CH), jnp.int32),
     pltpu.VMEM((NIDX, CH), jnp.int32)]
    + [pltpu.VMEM((CH, HALF), jnp.float32)] * NBUF
    + [pltpu.VMEM_SHARED((NP, HALF), jnp.float32)]
    + [pltpu.SemaphoreType.DMA] * (NIDX + 2 * NBUF)
)


def _unpack_pipe(rest):
    srcr, dstr = rest[0], rest[1]
    bufs = rest[2:2 + NBUF]
    acc = rest[2 + NBUF]
    sems = rest[3 + NBUF:]
    isems = sems[:NIDX]
    gsems = sems[NIDX:NIDX + NBUF]
    ssems = sems[NIDX + NBUF:]
    return srcr, dstr, bufs, acc, isems, gsems, ssems


@functools.partial(
    pl.kernel, mesh=_mesh,
    out_type=jax.ShapeDtypeStruct((2, NP, HALF), jnp.float32),
    scratch_types=list(_PIPE_SCRATCH),
)
def _prop1_kernel(xs_hbm, src_hbm, dst_hbm, out_hbm, *rest):
    srcr, dstr, bufs, acc, isems, gsems, ssems = _unpack_pipe(rest)
    c = lax.axis_index("c")
    s = lax.axis_index("s")
    pltpu.sync_copy(xs_hbm.at[pl.ds(c * NP + s * ROWS_T, ROWS_T)],
                    acc.at[pl.ds(s * ROWS_T, ROWS_T)])
    plsc.subcore_barrier()
    _edge_pipeline(xs_hbm,
                   lambda j: src_hbm.at[s, j],
                   lambda j: dst_hbm.at[s, j],
                   srcr, dstr, bufs, isems, gsems, ssems, acc, EP // 16 // CH,
                   src_off=c * NP)
    plsc.subcore_barrier()
    pltpu.sync_copy(acc.at[pl.ds(s * ROWS_T, ROWS_T)],
                    out_hbm.at[c, pl.ds(s * ROWS_T, ROWS_T)])


@functools.partial(
    pl.kernel, mesh=_mesh,
    out_type=jax.ShapeDtypeStruct((2, NP, HALF), jnp.float32),
    scratch_types=list(_PIPE_SCRATCH),
)
def _prop2_kernel(zs_hbm, src_hbm, dst_hbm, out_hbm, *rest):
    srcr, dstr, bufs, acc, isems, gsems, ssems = _unpack_pipe(rest)
    c = lax.axis_index("c")
    s = lax.axis_index("s")
    w = c * 16 + s

    @pl.when(c == 0)
    def _():
        pltpu.sync_copy(zs_hbm.at[pl.ds(s * ROWS_T, ROWS_T)],
                        acc.at[pl.ds(s * ROWS_T, ROWS_T)])

    @pl.when(c == 1)
    def _():
        _fill(bufs[0], 0.0)
        for i in range(ROWS_T // CH):
            pltpu.sync_copy(bufs[0], acc.at[pl.ds(s * ROWS_T + i * CH, CH)])

    plsc.subcore_barrier()
    _edge_pipeline(zs_hbm,
                   lambda j: src_hbm.at[w, j],
                   lambda j: dst_hbm.at[w, j],
                   srcr, dstr, bufs, isems, gsems, ssems, acc, EP // 32 // CH)
    plsc.subcore_barrier()
    pltpu.sync_copy(acc.at[pl.ds(s * ROWS_T, ROWS_T)],
                    out_hbm.at[c, pl.ds(s * ROWS_T, ROWS_T)])


_RB = 1024


def _tc1_body(degp_ref, x_ref, dis_ref, xs_ref):
    deg = degp_ref[0, :, 0:1] + degp_ref[1, :, 0:1] + 1.0
    dis = lax.rsqrt(deg)
    dis_ref[...] = dis
    xs_ref[0] = x_ref[:, 0:HALF] * dis
    xs_ref[1] = x_ref[:, HALF:D_IN] * dis


def _tc1(degp, x_pad):
    grid = (NP // _RB,)
    return pl.pallas_call(
        _tc1_body,
        grid=grid,
        in_specs=[
            pl.BlockSpec((2, _RB, HALF), lambda i: (0, i, 0)),
            pl.BlockSpec((_RB, D_IN), lambda i: (i, 0)),
        ],
        out_specs=[
            pl.BlockSpec((_RB, 1), lambda i: (i, 0)),
            pl.BlockSpec((2, _RB, HALF), lambda i: (0, i, 0)),
        ],
        out_shape=[
            jax.ShapeDtypeStruct((NP, 1), jnp.float32),
            jax.ShapeDtypeStruct((2, NP, HALF), jnp.float32),
        ],
    )(degp, x_pad)


def _tc2_body(p1_ref, dis_ref, w1_ref, b1_ref, w2_ref, zs_ref):
    y = jnp.dot(p1_ref[0], w1_ref[0], preferred_element_type=jnp.float32)
    y = y + jnp.dot(p1_ref[1], w1_ref[1], preferred_element_type=jnp.float32)
    h = jnp.maximum(y * dis_ref[...] + b1_ref[...], 0.0)
    zs = jnp.dot(h, w2_ref[...],
                 preferred_element_type=jnp.float32) * dis_ref[...]
    zs_ref[...] = jnp.concatenate(
        [zs, jnp.zeros((_RB, HALF - D_OUT), jnp.float32)], axis=1)


def _tc2(p1, dis, W1r, b1r, W2):
    grid = (NP // _RB,)
    return pl.pallas_call(
        _tc2_body,
        grid=grid,
        in_specs=[
            pl.BlockSpec((2, _RB, HALF), lambda i: (0, i, 0)),
            pl.BlockSpec((_RB, 1), lambda i: (i, 0)),
            pl.BlockSpec((2, HALF, D_HID), lambda i: (0, 0, 0)),
            pl.BlockSpec((1, D_HID), lambda i: (0, 0)),
            pl.BlockSpec((D_HID, D_OUT), lambda i: (0, 0)),
        ],
        out_specs=pl.BlockSpec((_RB, HALF), lambda i: (i, 0)),
        out_shape=jax.ShapeDtypeStruct((NP, HALF), jnp.float32),
    )(p1, dis, W1r, b1r, W2)


_RB3 = 1000


def _tc3_body(p2_ref, dis_ref, b2_ref, out_ref):
    out_ref[...] = ((p2_ref[0, :, 0:D_OUT] + p2_ref[1, :, 0:D_OUT])
                    * dis_ref[...] + b2_ref[...])


def _tc3(p2, dis, b2r):
    grid = (N // _RB3,)
    return pl.pallas_call(
        _tc3_body,
        grid=grid,
        in_specs=[
            pl.BlockSpec((2, _RB3, HALF), lambda i: (0, i, 0)),
            pl.BlockSpec((_RB3, 1), lambda i: (i, 0)),
            pl.BlockSpec((1, D_OUT), lambda i: (0, 0)),
        ],
        out_specs=pl.BlockSpec((_RB3, D_OUT), lambda i: (i, 0)),
        out_shape=jax.ShapeDtypeStruct((N, D_OUT), jnp.float32),
    )(p2, dis, b2r)


def kernel(x, edge_index, W1, b1, W2, b2):
    src = edge_index[0].astype(jnp.int32)
    dst = edge_index[1].astype(jnp.int32)

    pad_ids = N + (jnp.arange(EP - E, dtype=jnp.int32) % 16)
    src_p = jnp.concatenate([src, pad_ids])
    dst_p = jnp.concatenate([dst, pad_ids])

    src16 = src_p.reshape(16, EP // 16 // CH, CH)
    dst16 = dst_p.reshape(16, EP // 16 // CH, CH)
    src32 = src_p.reshape(32, EP // 32 // CH, CH)
    dst32 = dst_p.reshape(32, EP // 32 // CH, CH)

    x_pad = jnp.pad(x, ((0, NP - N), (0, 0)))

    degp = _deg_kernel(dst32)

    dis, xs = _tc1(degp, x_pad)
    xs_flat = xs.reshape(2 * NP, HALF)

    p1 = _prop1_kernel(xs_flat, src16, dst16)

    W1r = W1.reshape(2, HALF, D_HID)
    zs = _tc2(p1, dis, W1r, b1.reshape(1, D_HID), W2)

    p2 = _prop2_kernel(zs, src32, dst32)

    return _tc3(p2, dis, b2.reshape(1, D_OUT))


# speedup vs baseline: 23.3808x; 1.1127x over previous
import functools

import jax
import jax.numpy as jnp
from jax import lax
from jax.experimental import pallas as pl
from jax.experimental.pallas import tpu as pltpu
from jax.experimental.pallas import tpu_sc as plsc

N = 10000
NP = 10240
E = 160000
EP = 163840
D_IN = 256
D_HID = 512
D_OUT = 64
HALF = 128
CH = 128
ROWS_T = NP // 16

_mesh = plsc.VectorSubcoreMesh(core_axis_name="c", subcore_axis_name="s")


def _fill(buf_v, val, width=HALF):
    def frow(r, carry):
        for k in range(width // 16):
            buf_v[r, pl.ds(k * 16, 16)] = jnp.full((16,), val, jnp.float32)
        return carry
    lax.fori_loop(0, CH, frow, 0)


_UNTILED = pltpu.CompilerParams(use_tc_tiling_on_sc=False)


@functools.partial(
    pl.kernel, mesh=_mesh,
    out_type=jax.ShapeDtypeStruct((2, NP, 16), jnp.float32),
    scratch_types=[
        pltpu.VMEM((EP // 32 // CH, CH), jnp.int32),
        pltpu.VMEM((CH, 16), jnp.float32),
        pltpu.VMEM_SHARED((NP, 16), jnp.float32),
        pltpu.SemaphoreType.DMA,
    ],
    compiler_params=_UNTILED,
)
def _deg_kernel(dst_hbm, out_hbm, idx_v, buf_v, acc, sem):
    c = lax.axis_index("c")
    s = lax.axis_index("s")
    w = c * 16 + s
    pltpu.sync_copy(dst_hbm.at[w], idx_v)

    _fill(buf_v, 0.0, width=16)
    for i in range(ROWS_T // CH):
        pltpu.sync_copy(buf_v, acc.at[pl.ds(s * ROWS_T + i * CH, CH)])
    _fill(buf_v, 1.0, width=16)
    plsc.subcore_barrier()

    def body(j, carry):
        pltpu.async_copy(buf_v, acc.at[idx_v.at[j]], sem, add=True)
        return carry

    lax.fori_loop(0, EP // 32 // CH, body, 0)

    def drain(j, carry):
        pltpu.make_async_copy(buf_v, acc.at[idx_v.at[j]], sem).wait()
        return carry

    lax.fori_loop(0, EP // 32 // CH, drain, 0)
    plsc.subcore_barrier()
    pltpu.sync_copy(acc.at[pl.ds(s * ROWS_T, ROWS_T)],
                    out_hbm.at[c, pl.ds(s * ROWS_T, ROWS_T)])


NBUF = 2
NIDX = 4


def _edge_pipeline(table_hbm, src_row, dst_row, srcr, dstr, bufs,
                   isems, gsems, ssems, acc, n_chunks, src_off=None):
    for k in range(NBUF):
        pltpu.async_copy(src_row(k), srcr.at[k], isems[k])
        pltpu.async_copy(dst_row(k), dstr.at[k], isems[k])

    def chunk(j, b, q, qn):
        @pl.when(j >= NBUF)
        def _():
            pltpu.make_async_copy(bufs[b], acc.at[dstr.at[q]], ssems[b]).wait()

        @pl.when(j + NBUF < n_chunks)
        def _():
            pltpu.async_copy(src_row(j + NBUF), srcr.at[qn], isems[qn])
            pltpu.async_copy(dst_row(j + NBUF), dstr.at[qn], isems[qn])

        pltpu.make_async_copy(src_row(j), srcr.at[q], isems[q]).wait()
        pltpu.make_async_copy(dst_row(j), dstr.at[q], isems[q]).wait()
        if src_off is not None:
            for k in range(CH // 16):
                srcr[q, pl.ds(16 * k, 16)] = srcr[q, pl.ds(16 * k, 16)] + src_off
        pltpu.async_copy(table_hbm.at[srcr.at[q]], bufs[b], gsems[b])
        pltpu.make_async_copy(table_hbm.at[srcr.at[q]], bufs[b], gsems[b]).wait()
        pltpu.async_copy(bufs[b], acc.at[dstr.at[q]], ssems[b], add=True)

    def group(g, carry):
        for u in range(NIDX):
            chunk(g * NIDX + u, u % NBUF, u, (u + NBUF) % NIDX)
        return carry

    lax.fori_loop(0, n_chunks // NIDX, group, 0)
    for b in range(NBUF):
        pltpu.make_async_copy(bufs[b], acc.at[pl.ds(0, CH)], ssems[b]).wait()


def _pipe_scratch(width):
    return (
        [pltpu.VMEM((NIDX, CH), jnp.int32),
         pltpu.VMEM((NIDX, CH), jnp.int32)]
        + [pltpu.VMEM((CH, width), jnp.float32)] * NBUF
        + [pltpu.VMEM_SHARED((NP, width), jnp.float32)]
        + [pltpu.SemaphoreType.DMA] * (NIDX + 2 * NBUF)
    )


def _unpack_pipe(rest):
    srcr, dstr = rest[0], rest[1]
    bufs = rest[2:2 + NBUF]
    acc = rest[2 + NBUF]
    sems = rest[3 + NBUF:]
    isems = sems[:NIDX]
    gsems = sems[NIDX:NIDX + NBUF]
    ssems = sems[NIDX + NBUF:]
    return srcr, dstr, bufs, acc, isems, gsems, ssems


@functools.partial(
    pl.kernel, mesh=_mesh,
    out_type=jax.ShapeDtypeStruct((2, NP, HALF), jnp.float32),
    scratch_types=_pipe_scratch(HALF),
)
def _prop1_kernel(xs_hbm, src_hbm, dst_hbm, out_hbm, *rest):
    srcr, dstr, bufs, acc, isems, gsems, ssems = _unpack_pipe(rest)
    c = lax.axis_index("c")
    s = lax.axis_index("s")
    pltpu.sync_copy(xs_hbm.at[pl.ds(c * NP + s * ROWS_T, ROWS_T)],
                    acc.at[pl.ds(s * ROWS_T, ROWS_T)])
    plsc.subcore_barrier()
    _edge_pipeline(xs_hbm,
                   lambda j: src_hbm.at[s, j],
                   lambda j: dst_hbm.at[s, j],
                   srcr, dstr, bufs, isems, gsems, ssems, acc, EP // 16 // CH,
                   src_off=c * NP)
    plsc.subcore_barrier()
    pltpu.sync_copy(acc.at[pl.ds(s * ROWS_T, ROWS_T)],
                    out_hbm.at[c, pl.ds(s * ROWS_T, ROWS_T)])


@functools.partial(
    pl.kernel, mesh=_mesh,
    out_type=jax.ShapeDtypeStruct((2, NP, D_OUT), jnp.float32),
    scratch_types=_pipe_scratch(D_OUT),
    compiler_params=_UNTILED,
)
def _prop2_kernel(zs_hbm, src_hbm, dst_hbm, out_hbm, *rest):
    srcr, dstr, bufs, acc, isems, gsems, ssems = _unpack_pipe(rest)
    c = lax.axis_index("c")
    s = lax.axis_index("s")
    w = c * 16 + s

    @pl.when(c == 0)
    def _():
        pltpu.sync_copy(zs_hbm.at[pl.ds(s * ROWS_T, ROWS_T)],
                        acc.at[pl.ds(s * ROWS_T, ROWS_T)])

    @pl.when(c == 1)
    def _():
        _fill(bufs[0], 0.0, width=D_OUT)
        for i in range(ROWS_T // CH):
            pltpu.sync_copy(bufs[0], acc.at[pl.ds(s * ROWS_T + i * CH, CH)])

    plsc.subcore_barrier()
    _edge_pipeline(zs_hbm,
                   lambda j: src_hbm.at[w, j],
                   lambda j: dst_hbm.at[w, j],
                   srcr, dstr, bufs, isems, gsems, ssems, acc, EP // 32 // CH)
    plsc.subcore_barrier()
    pltpu.sync_copy(acc.at[pl.ds(s * ROWS_T, ROWS_T)],
                    out_hbm.at[c, pl.ds(s * ROWS_T, ROWS_T)])


_RB = 1024


def _tc1_body(degp_ref, x_ref, dis_ref, xs_ref):
    deg = degp_ref[0, :, 0:1] + degp_ref[1, :, 0:1] + 1.0
    dis = lax.rsqrt(deg)
    dis_ref[...] = dis
    xs_ref[0] = x_ref[:, 0:HALF] * dis
    xs_ref[1] = x_ref[:, HALF:D_IN] * dis


def _tc1(degp, x_pad):
    grid = (NP // _RB,)
    return pl.pallas_call(
        _tc1_body,
        grid=grid,
        in_specs=[
            pl.BlockSpec((2, _RB, 16), lambda i: (0, i, 0)),
            pl.BlockSpec((_RB, D_IN), lambda i: (i, 0)),
        ],
        out_specs=[
            pl.BlockSpec((_RB, 1), lambda i: (i, 0)),
            pl.BlockSpec((2, _RB, HALF), lambda i: (0, i, 0)),
        ],
        out_shape=[
            jax.ShapeDtypeStruct((NP, 1), jnp.float32),
            jax.ShapeDtypeStruct((2, NP, HALF), jnp.float32),
        ],
    )(degp, x_pad)


def _tc2_body(p1_ref, dis_ref, w1_ref, b1_ref, w2_ref, zs_ref):
    y = jnp.dot(p1_ref[0], w1_ref[0], preferred_element_type=jnp.float32)
    y = y + jnp.dot(p1_ref[1], w1_ref[1], preferred_element_type=jnp.float32)
    h = jnp.maximum(y * dis_ref[...] + b1_ref[...], 0.0)
    zs_ref[...] = jnp.dot(h, w2_ref[...],
                          preferred_element_type=jnp.float32) * dis_ref[...]


def _tc2(p1, dis, W1r, b1r, W2):
    grid = (NP // _RB,)
    return pl.pallas_call(
        _tc2_body,
        grid=grid,
        in_specs=[
            pl.BlockSpec((2, _RB, HALF), lambda i: (0, i, 0)),
            pl.BlockSpec((_RB, 1), lambda i: (i, 0)),
            pl.BlockSpec((2, HALF, D_HID), lambda i: (0, 0, 0)),
            pl.BlockSpec((1, D_HID), lambda i: (0, 0)),
            pl.BlockSpec((D_HID, D_OUT), lambda i: (0, 0)),
        ],
        out_specs=pl.BlockSpec((_RB, D_OUT), lambda i: (i, 0)),
        out_shape=jax.ShapeDtypeStruct((NP, D_OUT), jnp.float32),
    )(p1, dis, W1r, b1r, W2)


_RB3 = 1000


def _tc3_body(p2_ref, dis_ref, b2_ref, out_ref):
    out_ref[...] = ((p2_ref[0] + p2_ref[1]) * dis_ref[...] + b2_ref[...])


def _tc3(p2, dis, b2r):
    grid = (N // _RB3,)
    return pl.pallas_call(
        _tc3_body,
        grid=grid,
        in_specs=[
            pl.BlockSpec((2, _RB3, D_OUT), lambda i: (0, i, 0)),
            pl.BlockSpec((_RB3, 1), lambda i: (i, 0)),
            pl.BlockSpec((1, D_OUT), lambda i: (0, 0)),
        ],
        out_specs=pl.BlockSpec((_RB3, D_OUT), lambda i: (i, 0)),
        out_shape=jax.ShapeDtypeStruct((N, D_OUT), jnp.float32),
    )(p2, dis, b2r)


def kernel(x, edge_index, W1, b1, W2, b2):
    src = edge_index[0].astype(jnp.int32)
    dst = edge_index[1].astype(jnp.int32)

    pad_ids = N + (jnp.arange(EP - E, dtype=jnp.int32) % 16)
    src_p = jnp.concatenate([src, pad_ids])
    dst_p = jnp.concatenate([dst, pad_ids])

    src16 = src_p.reshape(16, EP // 16 // CH, CH)
    dst16 = dst_p.reshape(16, EP // 16 // CH, CH)
    src32 = src_p.reshape(32, EP // 32 // CH, CH)
    dst32 = dst_p.reshape(32, EP // 32 // CH, CH)

    x_pad = jnp.pad(x, ((0, NP - N), (0, 0)))

    degp = _deg_kernel(dst32)

    dis, xs = _tc1(degp, x_pad)
    xs_flat = xs.reshape(2 * NP, HALF)

    p1 = _prop1_kernel(xs_flat, src16, dst16)

    W1r = W1.reshape(2, HALF, D_HID)
    zs = _tc2(p1, dis, W1r, b1.reshape(1, D_HID), W2)

    p2 = _prop2_kernel(zs, src32, dst32)

    return _tc3(p2, dis, b2.reshape(1, D_OUT))


# speedup vs baseline: 23.4607x; 1.0034x over previous
import functools

import jax
import jax.numpy as jnp
from jax import lax
from jax.experimental import pallas as pl
from jax.experimental.pallas import tpu as pltpu
from jax.experimental.pallas import tpu_sc as plsc

N = 10000
NP = 10240
E = 160000
EP = 163840
D_IN = 256
D_HID = 512
D_OUT = 64
HALF = 128
CH = 128
ROWS_T = NP // 16

_mesh = plsc.VectorSubcoreMesh(core_axis_name="c", subcore_axis_name="s")


def _fill(buf_v, val, width=HALF, dtype=jnp.float32):
    vw = 32 if dtype == jnp.bfloat16 else 16
    def frow(r, carry):
        for k in range(width // vw):
            buf_v[r, pl.ds(k * vw, vw)] = jnp.full((vw,), val, dtype)
        return carry
    lax.fori_loop(0, CH, frow, 0)


_UNTILED = pltpu.CompilerParams(use_tc_tiling_on_sc=False)


@functools.partial(
    pl.kernel, mesh=_mesh,
    out_type=jax.ShapeDtypeStruct((2, NP, 16), jnp.float32),
    scratch_types=[
        pltpu.VMEM((EP // 32 // CH, CH), jnp.int32),
        pltpu.VMEM((CH, 16), jnp.float32),
        pltpu.VMEM_SHARED((NP, 16), jnp.float32),
        pltpu.SemaphoreType.DMA,
    ],
    compiler_params=_UNTILED,
)
def _deg_kernel(dst_hbm, out_hbm, idx_v, buf_v, acc, sem):
    c = lax.axis_index("c")
    s = lax.axis_index("s")
    w = c * 16 + s
    pltpu.sync_copy(dst_hbm.at[w], idx_v)

    _fill(buf_v, 0.0, width=16)
    for i in range(ROWS_T // CH):
        pltpu.sync_copy(buf_v, acc.at[pl.ds(s * ROWS_T + i * CH, CH)])
    _fill(buf_v, 1.0, width=16)
    plsc.subcore_barrier()

    def body(j, carry):
        pltpu.async_copy(buf_v, acc.at[idx_v.at[j]], sem, add=True)
        return carry

    lax.fori_loop(0, EP // 32 // CH, body, 0)

    def drain(j, carry):
        pltpu.make_async_copy(buf_v, acc.at[idx_v.at[j]], sem).wait()
        return carry

    lax.fori_loop(0, EP // 32 // CH, drain, 0)
    plsc.subcore_barrier()
    pltpu.sync_copy(acc.at[pl.ds(s * ROWS_T, ROWS_T)],
                    out_hbm.at[c, pl.ds(s * ROWS_T, ROWS_T)])


NBUF = 2
NIDX = 4


def _edge_pipeline(table_hbm, src_row, dst_row, srcr, dstr, bufs,
                   isems, gsems, ssems, acc, n_chunks, src_off=None):
    for k in range(NBUF):
        pltpu.async_copy(src_row(k), srcr.at[k], isems[k])
        pltpu.async_copy(dst_row(k), dstr.at[k], isems[k])

    def chunk(j, b, q, qn):
        @pl.when(j >= NBUF)
        def _():
            pltpu.make_async_copy(bufs[b], acc.at[dstr.at[q]], ssems[b]).wait()

        @pl.when(j + NBUF < n_chunks)
        def _():
            pltpu.async_copy(src_row(j + NBUF), srcr.at[qn], isems[qn])
            pltpu.async_copy(dst_row(j + NBUF), dstr.at[qn], isems[qn])

        pltpu.make_async_copy(src_row(j), srcr.at[q], isems[q]).wait()
        pltpu.make_async_copy(dst_row(j), dstr.at[q], isems[q]).wait()
        if src_off is not None:
            for k in range(CH // 16):
                srcr[q, pl.ds(16 * k, 16)] = srcr[q, pl.ds(16 * k, 16)] + src_off
        pltpu.async_copy(table_hbm.at[srcr.at[q]], bufs[b], gsems[b])
        pltpu.make_async_copy(table_hbm.at[srcr.at[q]], bufs[b], gsems[b]).wait()
        pltpu.async_copy(bufs[b], acc.at[dstr.at[q]], ssems[b], add=True)

    def group(g, carry):
        for u in range(NIDX):
            chunk(g * NIDX + u, u % NBUF, u, (u + NBUF) % NIDX)
        return carry

    lax.fori_loop(0, n_chunks // NIDX, group, 0)
    for b in range(NBUF):
        pltpu.make_async_copy(bufs[b], acc.at[pl.ds(0, CH)], ssems[b]).wait()


def _pipe_scratch(width, dtype=jnp.float32):
    return (
        [pltpu.VMEM((NIDX, CH), jnp.int32),
         pltpu.VMEM((NIDX, CH), jnp.int32)]
        + [pltpu.VMEM((CH, width), dtype)] * NBUF
        + [pltpu.VMEM_SHARED((NP, width), dtype)]
        + [pltpu.SemaphoreType.DMA] * (NIDX + 2 * NBUF)
    )


def _unpack_pipe(rest):
    srcr, dstr = rest[0], rest[1]
    bufs = rest[2:2 + NBUF]
    acc = rest[2 + NBUF]
    sems = rest[3 + NBUF:]
    isems = sems[:NIDX]
    gsems = sems[NIDX:NIDX + NBUF]
    ssems = sems[NIDX + NBUF:]
    return srcr, dstr, bufs, acc, isems, gsems, ssems


@functools.partial(
    pl.kernel, mesh=_mesh,
    out_type=jax.ShapeDtypeStruct((2, NP, HALF), jnp.bfloat16),
    scratch_types=_pipe_scratch(HALF, jnp.bfloat16),
    compiler_params=_UNTILED,
)
def _prop1_kernel(xs_hbm, src_hbm, dst_hbm, out_hbm, *rest):
    srcr, dstr, bufs, acc, isems, gsems, ssems = _unpack_pipe(rest)
    c = lax.axis_index("c")
    s = lax.axis_index("s")
    pltpu.sync_copy(xs_hbm.at[pl.ds(c * NP + s * ROWS_T, ROWS_T)],
                    acc.at[pl.ds(s * ROWS_T, ROWS_T)])
    plsc.subcore_barrier()
    _edge_pipeline(xs_hbm,
                   lambda j: src_hbm.at[s, j],
                   lambda j: dst_hbm.at[s, j],
                   srcr, dstr, bufs, isems, gsems, ssems, acc, EP // 16 // CH,
                   src_off=c * NP)
    plsc.subcore_barrier()
    pltpu.sync_copy(acc.at[pl.ds(s * ROWS_T, ROWS_T)],
                    out_hbm.at[c, pl.ds(s * ROWS_T, ROWS_T)])


@functools.partial(
    pl.kernel, mesh=_mesh,
    out_type=jax.ShapeDtypeStruct((2, NP, D_OUT), jnp.bfloat16),
    scratch_types=_pipe_scratch(D_OUT, jnp.bfloat16),
    compiler_params=_UNTILED,
)
def _prop2_kernel(zs_hbm, src_hbm, dst_hbm, out_hbm, *rest):
    srcr, dstr, bufs, acc, isems, gsems, ssems = _unpack_pipe(rest)
    c = lax.axis_index("c")
    s = lax.axis_index("s")
    w = c * 16 + s

    @pl.when(c == 0)
    def _():
        pltpu.sync_copy(zs_hbm.at[pl.ds(s * ROWS_T, ROWS_T)],
                        acc.at[pl.ds(s * ROWS_T, ROWS_T)])

    @pl.when(c == 1)
    def _():
        _fill(bufs[0], 0.0, width=D_OUT, dtype=jnp.bfloat16)
        for i in range(ROWS_T // CH):
            pltpu.sync_copy(bufs[0], acc.at[pl.ds(s * ROWS_T + i * CH, CH)])

    plsc.subcore_barrier()
    _edge_pipeline(zs_hbm,
                   lambda j: src_hbm.at[w, j],
                   lambda j: dst_hbm.at[w, j],
                   srcr, dstr, bufs, isems, gsems, ssems, acc, EP // 32 // CH)
    plsc.subcore_barrier()
    pltpu.sync_copy(acc.at[pl.ds(s * ROWS_T, ROWS_T)],
                    out_hbm.at[c, pl.ds(s * ROWS_T, ROWS_T)])


_RB = 1024


def _tc1_body(degp_ref, x_ref, dis_ref, xs_ref):
    deg = degp_ref[0, :, 0:1] + degp_ref[1, :, 0:1] + 1.0
    dis = lax.rsqrt(deg)
    dis_ref[...] = dis
    xs_ref[0] = (x_ref[:, 0:HALF] * dis).astype(jnp.bfloat16)
    xs_ref[1] = (x_ref[:, HALF:D_IN] * dis).astype(jnp.bfloat16)


def _tc1(degp, x_pad):
    grid = (NP // _RB,)
    return pl.pallas_call(
        _tc1_body,
        grid=grid,
        in_specs=[
            pl.BlockSpec((2, _RB, 16), lambda i: (0, i, 0)),
            pl.BlockSpec((_RB, D_IN), lambda i: (i, 0)),
        ],
        out_specs=[
            pl.BlockSpec((_RB, 1), lambda i: (i, 0)),
            pl.BlockSpec((2, _RB, HALF), lambda i: (0, i, 0)),
        ],
        out_shape=[
            jax.ShapeDtypeStruct((NP, 1), jnp.float32),
            jax.ShapeDtypeStruct((2, NP, HALF), jnp.bfloat16),
        ],
    )(degp, x_pad)


def _tc2_body(p1_ref, dis_ref, w1_ref, b1_ref, w2_ref, zs_ref):
    y = jnp.dot(p1_ref[0], w1_ref[0], preferred_element_type=jnp.float32)
    y = y + jnp.dot(p1_ref[1], w1_ref[1], preferred_element_type=jnp.float32)
    h = jnp.maximum(y * dis_ref[...] + b1_ref[...], 0.0)
    zs = jnp.dot(h.astype(jnp.bfloat16), w2_ref[...],
                 preferred_element_type=jnp.float32) * dis_ref[...]
    zs_ref[...] = zs.astype(jnp.bfloat16)


def _tc2(p1, dis, W1r, b1r, W2):
    grid = (NP // _RB,)
    return pl.pallas_call(
        _tc2_body,
        grid=grid,
        in_specs=[
            pl.BlockSpec((2, _RB, HALF), lambda i: (0, i, 0)),
            pl.BlockSpec((_RB, 1), lambda i: (i, 0)),
            pl.BlockSpec((2, HALF, D_HID), lambda i: (0, 0, 0)),
            pl.BlockSpec((1, D_HID), lambda i: (0, 0)),
            pl.BlockSpec((D_HID, D_OUT), lambda i: (0, 0)),
        ],
        out_specs=pl.BlockSpec((_RB, D_OUT), lambda i: (i, 0)),
        out_shape=jax.ShapeDtypeStruct((NP, D_OUT), jnp.bfloat16),
    )(p1, dis, W1r, b1r, W2)


_RB3 = 1000


def _tc3_body(p2_ref, dis_ref, b2_ref, out_ref):
    p2sum = p2_ref[0].astype(jnp.float32) + p2_ref[1].astype(jnp.float32)
    out_ref[...] = p2sum * dis_ref[...] + b2_ref[...]


def _tc3(p2, dis, b2r):
    grid = (N // _RB3,)
    return pl.pallas_call(
        _tc3_body,
        grid=grid,
        in_specs=[
            pl.BlockSpec((2, _RB3, D_OUT), lambda i: (0, i, 0)),
            pl.BlockSpec((_RB3, 1), lambda i: (i, 0)),
            pl.BlockSpec((1, D_OUT), lambda i: (0, 0)),
        ],
        out_specs=pl.BlockSpec((_RB3, D_OUT), lambda i: (i, 0)),
        out_shape=jax.ShapeDtypeStruct((N, D_OUT), jnp.float32),
    )(p2, dis, b2r)


def kernel(x, edge_index, W1, b1, W2, b2):
    src = edge_index[0].astype(jnp.int32)
    dst = edge_index[1].astype(jnp.int32)

    pad_ids = N + (jnp.arange(EP - E, dtype=jnp.int32) % 16)
    src_p = jnp.concatenate([src, pad_ids])
    dst_p = jnp.concatenate([dst, pad_ids])

    src16 = src_p.reshape(16, EP // 16 // CH, CH)
    dst16 = dst_p.reshape(16, EP // 16 // CH, CH)
    src32 = src_p.reshape(32, EP // 32 // CH, CH)
    dst32 = dst_p.reshape(32, EP // 32 // CH, CH)

    x_pad = jnp.pad(x, ((0, NP - N), (0, 0)))

    degp = _deg_kernel(dst32)

    dis, xs = _tc1(degp, x_pad)
    xs_flat = xs.reshape(2 * NP, HALF)

    p1 = _prop1_kernel(xs_flat, src16, dst16)

    W1r = W1.reshape(2, HALF, D_HID).astype(jnp.bfloat16)
    zs = _tc2(p1, dis, W1r, b1.reshape(1, D_HID), W2.astype(jnp.bfloat16))

    p2 = _prop2_kernel(zs, src32, dst32)

    return _tc3(p2, dis, b2.reshape(1, D_OUT))


# speedup vs baseline: 27.2816x; 1.1629x over previous
import functools

import jax
import jax.numpy as jnp
from jax import lax
from jax.experimental import pallas as pl
from jax.experimental.pallas import tpu as pltpu
from jax.experimental.pallas import tpu_sc as plsc

N = 10000
NP = 10240
E = 160000
EP = 163840
D_IN = 256
D_HID = 512
D_OUT = 64
HALF = 128
CH = 128
ROWS_T = NP // 16

_mesh = plsc.VectorSubcoreMesh(core_axis_name="c", subcore_axis_name="s")


def _fill(buf_v, val, width=HALF, dtype=jnp.float32):
    vw = 32 if dtype == jnp.bfloat16 else 16
    def frow(r, carry):
        for k in range(width // vw):
            buf_v[r, pl.ds(k * vw, vw)] = jnp.full((vw,), val, dtype)
        return carry
    lax.fori_loop(0, CH, frow, 0)


_UNTILED = pltpu.CompilerParams(use_tc_tiling_on_sc=False)


@functools.partial(
    pl.kernel, mesh=_mesh,
    out_type=jax.ShapeDtypeStruct((2, NP, 16), jnp.float32),
    scratch_types=[
        pltpu.VMEM((EP // 32 // CH, CH), jnp.int32),
        pltpu.VMEM((CH, 16), jnp.float32),
        pltpu.VMEM_SHARED((NP, 16), jnp.float32),
        pltpu.SemaphoreType.DMA,
    ],
    compiler_params=_UNTILED,
)
def _deg_kernel(dst_hbm, out_hbm, idx_v, buf_v, acc, sem):
    c = lax.axis_index("c")
    s = lax.axis_index("s")
    w = c * 16 + s
    pltpu.sync_copy(dst_hbm.at[w], idx_v)

    _fill(buf_v, 0.0, width=16)
    for i in range(ROWS_T // CH):
        pltpu.sync_copy(buf_v, acc.at[pl.ds(s * ROWS_T + i * CH, CH)])
    _fill(buf_v, 1.0, width=16)
    plsc.subcore_barrier()

    def body(j, carry):
        pltpu.async_copy(buf_v, acc.at[idx_v.at[j]], sem, add=True)
        return carry

    lax.fori_loop(0, EP // 32 // CH, body, 0)

    def drain(j, carry):
        pltpu.make_async_copy(buf_v, acc.at[idx_v.at[j]], sem).wait()
        return carry

    lax.fori_loop(0, EP // 32 // CH, drain, 0)
    plsc.subcore_barrier()
    pltpu.sync_copy(acc.at[pl.ds(s * ROWS_T, ROWS_T)],
                    out_hbm.at[c, pl.ds(s * ROWS_T, ROWS_T)])


NBUF = 2
NIDX = 4


def _edge_pipeline(table_hbm, src_row, dst_row, srcr, dstr, bufs,
                   isems, gsems, ssems, acc, n_chunks, src_off=None):
    for k in range(NBUF):
        pltpu.async_copy(src_row(k), srcr.at[k], isems[k])
        pltpu.async_copy(dst_row(k), dstr.at[k], isems[k])

    def issue(j, b, q, qn):
        @pl.when(j >= NBUF)
        def _():
            pltpu.make_async_copy(bufs[b], acc.at[dstr.at[q]], ssems[b]).wait()

        @pl.when(j + NBUF < n_chunks)
        def _():
            pltpu.async_copy(src_row(j + NBUF), srcr.at[qn], isems[qn])
            pltpu.async_copy(dst_row(j + NBUF), dstr.at[qn], isems[qn])

        pltpu.make_async_copy(src_row(j), srcr.at[q], isems[q]).wait()
        pltpu.make_async_copy(dst_row(j), dstr.at[q], isems[q]).wait()
        if src_off is not None:
            for k in range(CH // 16):
                srcr[q, pl.ds(16 * k, 16)] = srcr[q, pl.ds(16 * k, 16)] + src_off
        pltpu.async_copy(table_hbm.at[srcr.at[q]], bufs[b], gsems[b])

    def complete(j, b, q):
        pltpu.make_async_copy(table_hbm.at[srcr.at[q]], bufs[b], gsems[b]).wait()
        pltpu.async_copy(bufs[b], acc.at[dstr.at[q]], ssems[b], add=True)

    def group(g, carry):
        for u in range(NIDX):
            j = g * NIDX + u
            issue(j, u % NBUF, u, (u + NBUF) % NIDX)
            if u == 0:
                @pl.when(j >= 1)
                def _():
                    complete(j - 1, (NIDX - 1) % NBUF, NIDX - 1)
            else:
                complete(j - 1, (u - 1) % NBUF, u - 1)
        return carry

    lax.fori_loop(0, n_chunks // NIDX, group, 0)
    complete(n_chunks - 1, (NIDX - 1) % NBUF, NIDX - 1)
    for b in range(NBUF):
        pltpu.make_async_copy(bufs[b], acc.at[pl.ds(0, CH)], ssems[b]).wait()


def _pipe_scratch(width, dtype=jnp.float32):
    return (
        [pltpu.VMEM((NIDX, CH), jnp.int32),
         pltpu.VMEM((NIDX, CH), jnp.int32)]
        + [pltpu.VMEM((CH, width), dtype)] * NBUF
        + [pltpu.VMEM_SHARED((NP, width), dtype)]
        + [pltpu.SemaphoreType.DMA] * (NIDX + 2 * NBUF)
    )


def _unpack_pipe(rest):
    srcr, dstr = rest[0], rest[1]
    bufs = rest[2:2 + NBUF]
    acc = rest[2 + NBUF]
    sems = rest[3 + NBUF:]
    isems = sems[:NIDX]
    gsems = sems[NIDX:NIDX + NBUF]
    ssems = sems[NIDX + NBUF:]
    return srcr, dstr, bufs, acc, isems, gsems, ssems


@functools.partial(
    pl.kernel, mesh=_mesh,
    out_type=jax.ShapeDtypeStruct((2, NP, D_IN), jnp.bfloat16),
    scratch_types=_pipe_scratch(D_IN, jnp.bfloat16),
    compiler_params=_UNTILED,
)
def _prop1_kernel(xs_hbm, src_hbm, dst_hbm, out_hbm, *rest):
    srcr, dstr, bufs, acc, isems, gsems, ssems = _unpack_pipe(rest)
    c = lax.axis_index("c")
    s = lax.axis_index("s")
    w = c * 16 + s

    @pl.when(c == 0)
    def _():
        pltpu.sync_copy(xs_hbm.at[pl.ds(s * ROWS_T, ROWS_T)],
                        acc.at[pl.ds(s * ROWS_T, ROWS_T)])

    @pl.when(c == 1)
    def _():
        _fill(bufs[0], 0.0, width=D_IN, dtype=jnp.bfloat16)
        for i in range(ROWS_T // CH):
            pltpu.sync_copy(bufs[0], acc.at[pl.ds(s * ROWS_T + i * CH, CH)])

    plsc.subcore_barrier()
    _edge_pipeline(xs_hbm,
                   lambda j: src_hbm.at[w, j],
                   lambda j: dst_hbm.at[w, j],
                   srcr, dstr, bufs, isems, gsems, ssems, acc, EP // 32 // CH)
    plsc.subcore_barrier()
    pltpu.sync_copy(acc.at[pl.ds(s * ROWS_T, ROWS_T)],
                    out_hbm.at[c, pl.ds(s * ROWS_T, ROWS_T)])


@functools.partial(
    pl.kernel, mesh=_mesh,
    out_type=jax.ShapeDtypeStruct((2, NP, D_OUT), jnp.bfloat16),
    scratch_types=_pipe_scratch(D_OUT, jnp.bfloat16),
    compiler_params=_UNTILED,
)
def _prop2_kernel(zs_hbm, src_hbm, dst_hbm, out_hbm, *rest):
    srcr, dstr, bufs, acc, isems, gsems, ssems = _unpack_pipe(rest)
    c = lax.axis_index("c")
    s = lax.axis_index("s")
    w = c * 16 + s

    @pl.when(c == 0)
    def _():
        pltpu.sync_copy(zs_hbm.at[pl.ds(s * ROWS_T, ROWS_T)],
                        acc.at[pl.ds(s * ROWS_T, ROWS_T)])

    @pl.when(c == 1)
    def _():
        _fill(bufs[0], 0.0, width=D_OUT, dtype=jnp.bfloat16)
        for i in range(ROWS_T // CH):
            pltpu.sync_copy(bufs[0], acc.at[pl.ds(s * ROWS_T + i * CH, CH)])

    plsc.subcore_barrier()
    _edge_pipeline(zs_hbm,
                   lambda j: src_hbm.at[w, j],
                   lambda j: dst_hbm.at[w, j],
                   srcr, dstr, bufs, isems, gsems, ssems, acc, EP // 32 // CH)
    plsc.subcore_barrier()
    pltpu.sync_copy(acc.at[pl.ds(s * ROWS_T, ROWS_T)],
                    out_hbm.at[c, pl.ds(s * ROWS_T, ROWS_T)])


_RB = 1024


def _tc1_body(degp_ref, x_ref, dis_ref, xs_ref):
    deg = degp_ref[0, :, 0:1] + degp_ref[1, :, 0:1] + 1.0
    dis = lax.rsqrt(deg)
    dis_ref[...] = dis
    xs_ref[...] = (x_ref[...] * dis).astype(jnp.bfloat16)


def _tc1(degp, x_pad):
    grid = (NP // _RB,)
    return pl.pallas_call(
        _tc1_body,
        grid=grid,
        in_specs=[
            pl.BlockSpec((2, _RB, 16), lambda i: (0, i, 0)),
            pl.BlockSpec((_RB, D_IN), lambda i: (i, 0)),
        ],
        out_specs=[
            pl.BlockSpec((_RB, 1), lambda i: (i, 0)),
            pl.BlockSpec((_RB, D_IN), lambda i: (i, 0)),
        ],
        out_shape=[
            jax.ShapeDtypeStruct((NP, 1), jnp.float32),
            jax.ShapeDtypeStruct((NP, D_IN), jnp.bfloat16),
        ],
    )(degp, x_pad)


def _tc2_body(p1_ref, dis_ref, w1_ref, b1_ref, w2_ref, zs_ref):
    p1 = (p1_ref[0].astype(jnp.float32)
          + p1_ref[1].astype(jnp.float32)).astype(jnp.bfloat16)
    y = jnp.dot(p1, w1_ref[...], preferred_element_type=jnp.float32)
    h = jnp.maximum(y * dis_ref[...] + b1_ref[...], 0.0)
    zs = jnp.dot(h.astype(jnp.bfloat16), w2_ref[...],
                 preferred_element_type=jnp.float32) * dis_ref[...]
    zs_ref[...] = zs.astype(jnp.bfloat16)


def _tc2(p1, dis, W1r, b1r, W2):
    grid = (NP // _RB,)
    return pl.pallas_call(
        _tc2_body,
        grid=grid,
        in_specs=[
            pl.BlockSpec((2, _RB, D_IN), lambda i: (0, i, 0)),
            pl.BlockSpec((_RB, 1), lambda i: (i, 0)),
            pl.BlockSpec((D_IN, D_HID), lambda i: (0, 0)),
            pl.BlockSpec((1, D_HID), lambda i: (0, 0)),
            pl.BlockSpec((D_HID, D_OUT), lambda i: (0, 0)),
        ],
        out_specs=pl.BlockSpec((_RB, D_OUT), lambda i: (i, 0)),
        out_shape=jax.ShapeDtypeStruct((NP, D_OUT), jnp.bfloat16),
    )(p1, dis, W1r, b1r, W2)


_RB3 = 1000


def _tc3_body(p2_ref, dis_ref, b2_ref, out_ref):
    p2sum = p2_ref[0].astype(jnp.float32) + p2_ref[1].astype(jnp.float32)
    out_ref[...] = p2sum * dis_ref[...] + b2_ref[...]


def _tc3(p2, dis, b2r):
    grid = (N // _RB3,)
    return pl.pallas_call(
        _tc3_body,
        grid=grid,
        in_specs=[
            pl.BlockSpec((2, _RB3, D_OUT), lambda i: (0, i, 0)),
            pl.BlockSpec((_RB3, 1), lambda i: (i, 0)),
            pl.BlockSpec((1, D_OUT), lambda i: (0, 0)),
        ],
        out_specs=pl.BlockSpec((_RB3, D_OUT), lambda i: (i, 0)),
        out_shape=jax.ShapeDtypeStruct((N, D_OUT), jnp.float32),
    )(p2, dis, b2r)


def kernel(x, edge_index, W1, b1, W2, b2):
    src = edge_index[0].astype(jnp.int32)
    dst = edge_index[1].astype(jnp.int32)

    pad_ids = N + (jnp.arange(EP - E, dtype=jnp.int32) % 16)
    src_p = jnp.concatenate([src, pad_ids])
    dst_p = jnp.concatenate([dst, pad_ids])

    src32 = src_p.reshape(32, EP // 32 // CH, CH)
    dst32 = dst_p.reshape(32, EP // 32 // CH, CH)

    x_pad = jnp.pad(x, ((0, NP - N), (0, 0)))

    degp = _deg_kernel(dst32)

    dis, xs = _tc1(degp, x_pad)

    p1 = _prop1_kernel(xs, src32, dst32)

    zs = _tc2(p1, dis, W1.astype(jnp.bfloat16), b1.reshape(1, D_HID),
              W2.astype(jnp.bfloat16))

    p2 = _prop2_kernel(zs, src32, dst32)

    return _tc3(p2, dis, b2.reshape(1, D_OUT))
